# baseline (speedup 1.0000x reference)
import jax
import jax.numpy as jnp
from jax.experimental import pallas as pl
from jax.experimental.pallas import tpu as pltpu

B = 262144
K = 64
RBLK = 8192


def _body(gu_ref, gi_ref, bu_ref, bi_ref, mu_ref, out_ref):
    z = gu_ref[...] * gi_ref[...]
    s = jnp.sum(z, axis=1)
    out_ref[...] = s + bu_ref[...][:, 0] + bi_ref[...][:, 0] + mu_ref[0, 0]


def kernel(gu, gi, bu, bi, Mu):
    return pl.pallas_call(
        _body,
        grid=(B // RBLK,),
        in_specs=[
            pl.BlockSpec((RBLK, K), lambda i: (i, 0)),
            pl.BlockSpec((RBLK, K), lambda i: (i, 0)),
            pl.BlockSpec((RBLK, 1), lambda i: (i, 0)),
            pl.BlockSpec((RBLK, 1), lambda i: (i, 0)),
            pl.BlockSpec((1, 1), lambda i: (0, 0)),
        ],
        out_specs=pl.BlockSpec((RBLK,), lambda i: (i,)),
        out_shape=jax.ShapeDtypeStruct((B,), jnp.float32),
    )(gu, gi, bu, bi, Mu)


# speedup vs baseline: 10.7985x; 10.7985x over previous
import jax
import jax.numpy as jnp
from jax.experimental import pallas as pl
from jax.experimental.pallas import tpu as pltpu

B = 262144
K = 64
CB = 16384


def _body(gu_ref, gi_ref, bu_ref, bi_ref, mu_ref, out_ref):
    z = gu_ref[...] * gi_ref[...]
    out_ref[...] = jnp.sum(z, axis=0) + bu_ref[...] + bi_ref[...] + mu_ref[0, 0]


def kernel(gu, gi, bu, bi, Mu):
    gut = gu.T
    git = gi.T
    buf = bu.reshape(B)
    bif = bi.reshape(B)
    return pl.pallas_call(
        _body,
        grid=(B // CB,),
        in_specs=[
            pl.BlockSpec((K, CB), lambda i: (0, i)),
            pl.BlockSpec((K, CB), lambda i: (0, i)),
            pl.BlockSpec((CB,), lambda i: (i,)),
            pl.BlockSpec((CB,), lambda i: (i,)),
            pl.BlockSpec((1, 1), lambda i: (0, 0)),
        ],
        out_specs=pl.BlockSpec((CB,), lambda i: (i,)),
        out_shape=jax.ShapeDtypeStruct((B,), jnp.float32),
    )(gut, git, buf, bif, Mu)
